# Initial kernel scaffold; baseline (speedup 1.0000x reference)
import jax
import jax.numpy as jnp
from jax.experimental import pallas as pl


def kernel(tensor, labels, embed_table, v0):
    raise NotImplementedError("write your pallas kernel here")


# speedup vs baseline: 1.7050x; 1.7050x over previous
import jax
import jax.numpy as jnp
from jax.experimental import pallas as pl
from jax.experimental.pallas import tpu as pltpu

_NUM_ROWS = 1000
_ROW = 8192
_SUB = 64
_LANE = 128
_BATCH = 1024
_BB = 128


def _inv_sn_kernel(w_ref, v0_ref, inv_ref):
    w = w_ref[...]
    v0 = v0_ref[...]
    t1 = jnp.dot(w, v0, preferred_element_type=jnp.float32)
    n1 = jnp.sum(t1 * t1)
    t2 = jax.lax.dot_general(
        t1, w, (((0,), (0,)), ((), ())),
        preferred_element_type=jnp.float32)
    n2 = jnp.sum(t2 * t2)
    inv_ref[0, 0] = jnp.sqrt(n1 / n2)


def _cond_kernel(labels_ref, inv_ref, table_ref, tensor_ref, out_ref):
    i = pl.program_id(0)
    inv = inv_ref[0, 0]

    def body(j, _):
        lab = labels_ref[i * _BB + j]
        out_ref[pl.ds(j, 1)] = (
            tensor_ref[pl.ds(j, 1)] + table_ref[pl.ds(lab, 1)] * inv)
        return 0

    jax.lax.fori_loop(0, _BB, body, 0, unroll=True)


def kernel(tensor, labels, embed_table, v0):
    labels = labels.astype(jnp.int32)
    inv_sn = pl.pallas_call(
        _inv_sn_kernel,
        in_specs=[
            pl.BlockSpec(memory_space=pltpu.VMEM),
            pl.BlockSpec(memory_space=pltpu.VMEM),
        ],
        out_specs=pl.BlockSpec(memory_space=pltpu.SMEM),
        out_shape=jax.ShapeDtypeStruct((1, 1), jnp.float32),
    )(embed_table, v0)

    table3 = embed_table.reshape(_NUM_ROWS, _SUB, _LANE)
    tensor3 = tensor.reshape(_BATCH, _SUB, _LANE)

    out = pl.pallas_call(
        _cond_kernel,
        grid=(_BATCH // _BB,),
        in_specs=[
            pl.BlockSpec(memory_space=pltpu.SMEM),
            pl.BlockSpec(memory_space=pltpu.SMEM),
            pl.BlockSpec((_NUM_ROWS, _SUB, _LANE), lambda i: (0, 0, 0)),
            pl.BlockSpec((_BB, _SUB, _LANE), lambda i: (i, 0, 0)),
        ],
        out_specs=pl.BlockSpec((_BB, _SUB, _LANE), lambda i: (i, 0, 0)),
        out_shape=jax.ShapeDtypeStruct((_BATCH, _SUB, _LANE), jnp.float32),
    )(labels, inv_sn, table3, tensor3)

    return out.reshape(tensor.shape)


# speedup vs baseline: 1.7056x; 1.0003x over previous
import jax
import jax.numpy as jnp
from jax.experimental import pallas as pl
from jax.experimental.pallas import tpu as pltpu

_NUM_ROWS = 1000
_ROW = 8192
_SUB = 64
_LANE = 128
_BATCH = 1024
_BB = 128


def _inv_sn_kernel(w_ref, v0_ref, inv_ref):
    w = w_ref[...]
    v0 = v0_ref[...]
    t1 = jnp.dot(w, v0, preferred_element_type=jnp.float32)
    n1 = jnp.sum(t1 * t1)
    t2 = jax.lax.dot_general(
        t1, w, (((0,), (0,)), ((), ())),
        preferred_element_type=jnp.float32)
    n2 = jnp.sum(t2 * t2)
    inv_ref[0, 0] = jnp.sqrt(n1 / n2)


def _cond_kernel(labels_ref, inv_ref, table_ref, tensor_ref, out_ref):
    i = pl.program_id(0)
    inv = inv_ref[0, 0]

    def body(j, _):
        lab = labels_ref[i * _BB + j]
        row = table_ref[pl.ds(lab, 1)].reshape(1, 8, 8, _LANE)
        out_ref[pl.ds(j, 1)] = tensor_ref[pl.ds(j, 1)] + row * inv
        return 0

    jax.lax.fori_loop(0, _BB, body, 0, unroll=True)


def kernel(tensor, labels, embed_table, v0):
    labels = labels.astype(jnp.int32)
    inv_sn = pl.pallas_call(
        _inv_sn_kernel,
        in_specs=[
            pl.BlockSpec(memory_space=pltpu.VMEM),
            pl.BlockSpec(memory_space=pltpu.VMEM),
        ],
        out_specs=pl.BlockSpec(memory_space=pltpu.SMEM),
        out_shape=jax.ShapeDtypeStruct((1, 1), jnp.float32),
    )(embed_table, v0)

    table3 = embed_table.reshape(_NUM_ROWS, _SUB, _LANE)

    out = pl.pallas_call(
        _cond_kernel,
        grid=(_BATCH // _BB,),
        in_specs=[
            pl.BlockSpec(memory_space=pltpu.SMEM),
            pl.BlockSpec(memory_space=pltpu.SMEM),
            pl.BlockSpec((_NUM_ROWS, _SUB, _LANE), lambda i: (0, 0, 0)),
            pl.BlockSpec((_BB, 8, 8, _LANE), lambda i: (i, 0, 0, 0)),
        ],
        out_specs=pl.BlockSpec((_BB, 8, 8, _LANE), lambda i: (i, 0, 0, 0)),
        out_shape=jax.ShapeDtypeStruct(tensor.shape, jnp.float32),
    )(labels, inv_sn, table3, tensor)

    return out
